# baseline (speedup 1.0000x reference)
import functools

import jax
import jax.numpy as jnp
from jax import lax
from jax.experimental import pallas as pl
from jax.experimental.pallas import tpu as pltpu
from jax.experimental.pallas import tpu_sc as plsc

NP = 10240
NCORES = 2
NSUB = 16
NTILES = NCORES * NSUB
CHUNK = 80
BN = 2560
LN = 2048


def _sc_degree(E):
    nchunks = E // NTILES // CHUNK
    rows_per_tile = NP // NSUB
    nz = rows_per_tile // CHUNK
    mesh = plsc.VectorSubcoreMesh(core_axis_name="c", subcore_axis_name="s")

    @functools.partial(
        pl.kernel,
        mesh=mesh,
        out_type=jax.ShapeDtypeStruct((NCORES * NP,), jnp.float32),
        compiler_params=pltpu.CompilerParams(use_tc_tiling_on_sc=False),
        scratch_types=[
            pltpu.VMEM((nchunks, CHUNK), jnp.int32),
            pltpu.VMEM((CHUNK,), jnp.float32),
            pltpu.VMEM((CHUNK,), jnp.float32),
            pltpu.SemaphoreType.DMA,
            pltpu.VMEM_SHARED((NP,), jnp.float32),
        ],
    )
    def deg_kernel(dst_hbm, out_hbm, didx, ones, stage, sem, acc):
        c = lax.axis_index("c")
        s = lax.axis_index("s")
        t = c * NSUB + s
        one = jnp.ones((16,), jnp.float32)
        zero = jnp.zeros((16,), jnp.float32)
        for j in range(CHUNK // 16):
            ones[pl.ds(j * 16, 16)] = one
            stage[pl.ds(j * 16, 16)] = zero

        def zbody(k, carry):
            pltpu.sync_copy(stage, acc.at[pl.ds(s * rows_per_tile + k * CHUNK, CHUNK)])
            return carry

        lax.fori_loop(0, nz, zbody, 0)
        plsc.subcore_barrier()
        pltpu.sync_copy(dst_hbm.at[t], didx)

        burst = 5

        def body(ii, carry):
            for b in range(burst):
                pltpu.async_copy(ones, acc.at[didx.at[burst * ii + b]], sem,
                                 add=True)
            for b in range(burst):
                pltpu.make_async_copy(ones, acc.at[didx.at[0]], sem).wait()
            return carry

        lax.fori_loop(0, nchunks // burst, body, 0)
        for k in range(nchunks - burst * (nchunks // burst)):
            pltpu.sync_copy(ones, acc.at[didx.at[burst * (nchunks // burst) + k]],
                            add=True)
        plsc.subcore_barrier()

        def obody(k, carry):
            off = s * rows_per_tile + k * CHUNK
            pltpu.sync_copy(acc.at[pl.ds(off, CHUNK)], stage)
            pltpu.sync_copy(stage, out_hbm.at[pl.ds(c * NP + off, CHUNK)])
            return carry

        lax.fori_loop(0, nz, obody, 0)

    return deg_kernel


def _sc_gather_scatter_rows(E, W):
    nper = E // NTILES
    CH = 128
    nmain = nper // CH
    tail = nper - nmain * CH
    rows_per_tile = NP // NSUB
    nz = rows_per_tile // CH
    NBUF = 4
    mesh = plsc.VectorSubcoreMesh(core_axis_name="c", subcore_axis_name="s")

    @functools.partial(
        pl.kernel,
        mesh=mesh,
        out_type=jax.ShapeDtypeStruct((NCORES, NP, W), jnp.float32),
        compiler_params=pltpu.CompilerParams(use_tc_tiling_on_sc=False),
        scratch_types=(
            [pltpu.VMEM((nmain, CH), jnp.int32),
             pltpu.VMEM((nmain, CH), jnp.int32),
             pltpu.VMEM((tail,), jnp.int32),
             pltpu.VMEM((tail,), jnp.int32),
             pltpu.VMEM((tail, W), jnp.float32)]
            + [pltpu.VMEM((CH, W), jnp.float32)] * (NBUF + 1)
            + [pltpu.SemaphoreType.DMA] * (2 * NBUF)
            + [pltpu.VMEM_SHARED((NP, W), jnp.float32)]
        ),
    )
    def gs_kernel(table_hbm, srcm_hbm, dstm_hbm, srct_hbm, dstt_hbm, out_hbm,
                  sidx, didx, sidxt, didxt, rowst,
                  r0, r1, r2, r3, stage,
                  g0, g1, g2, g3,
                  s0, s1, s2, s3, acc):
        rows = (r0, r1, r2, r3)
        gsem = (g0, g1, g2, g3)
        ssem = (s0, s1, s2, s3)
        nfull = nmain // NBUF
        rem = nmain - nfull * NBUF
        c = lax.axis_index("c")
        s = lax.axis_index("s")
        t = c * NSUB + s
        zero = jnp.zeros((16,), jnp.float32)

        def zrow(i, carry):
            for j in range(W // 16):
                stage[i, pl.ds(j * 16, 16)] = zero
            return carry

        lax.fori_loop(0, CH, zrow, 0)

        def zbody(k, carry):
            pltpu.sync_copy(stage, acc.at[pl.ds(s * rows_per_tile + k * CH, CH)])
            return carry

        lax.fori_loop(0, nz, zbody, 0)
        plsc.subcore_barrier()
        pltpu.sync_copy(srcm_hbm.at[t], sidx)
        pltpu.sync_copy(dstm_hbm.at[t], didx)
        pltpu.sync_copy(srct_hbm.at[t], sidxt)
        pltpu.sync_copy(dstt_hbm.at[t], didxt)

        for b in range(NBUF):
            pltpu.async_copy(table_hbm.at[sidx.at[b]], rows[b], gsem[b])
        for b in range(NBUF):
            pltpu.make_async_copy(table_hbm.at[sidx.at[b]], rows[b], gsem[b]).wait()
            pltpu.async_copy(rows[b], acc.at[didx.at[b]], ssem[b], add=True)

        def body(ii, carry):
            base = ii * NBUF
            for b in range(NBUF):
                pltpu.make_async_copy(rows[b], acc.at[didx.at[0]], ssem[b]).wait()
                pltpu.async_copy(table_hbm.at[sidx.at[base + b]], rows[b], gsem[b])
            for b in range(NBUF):
                pltpu.make_async_copy(table_hbm.at[sidx.at[base + b]],
                                      rows[b], gsem[b]).wait()
                pltpu.async_copy(rows[b], acc.at[didx.at[base + b]], ssem[b],
                                 add=True)
            return carry

        lax.fori_loop(1, nfull, body, 0)
        for b in range(NBUF):
            pltpu.make_async_copy(rows[b], acc.at[didx.at[0]], ssem[b]).wait()
        for k in range(rem):
            i = nfull * NBUF + k
            pltpu.async_copy(table_hbm.at[sidx.at[i]], rows[k], gsem[k])
        for k in range(rem):
            i = nfull * NBUF + k
            pltpu.make_async_copy(table_hbm.at[sidx.at[i]], rows[k], gsem[k]).wait()
            pltpu.async_copy(rows[k], acc.at[didx.at[i]], ssem[k], add=True)
        pltpu.async_copy(table_hbm.at[sidxt], rowst, gsem[NBUF - 1])
        pltpu.make_async_copy(table_hbm.at[sidxt], rowst, gsem[NBUF - 1]).wait()
        pltpu.sync_copy(rowst, acc.at[didxt], add=True)
        for k in range(rem):
            pltpu.make_async_copy(rows[k], acc.at[didx.at[0]], ssem[k]).wait()
        plsc.subcore_barrier()

        def obody(k, carry):
            off = s * rows_per_tile + k * CH
            pltpu.sync_copy(acc.at[pl.ds(off, CH)], stage)
            pltpu.sync_copy(stage, out_hbm.at[c, pl.ds(off, CH)])
            return carry

        lax.fori_loop(0, nz, obody, 0)

    return gs_kernel


def _sc_gather_scatter_elems(E):
    nchunks = E // NTILES // CHUNK
    rows_per_tile = NP // NSUB
    nz = rows_per_tile // CHUNK
    mesh = plsc.VectorSubcoreMesh(core_axis_name="c", subcore_axis_name="s")

    @functools.partial(
        pl.kernel,
        mesh=mesh,
        out_type=[jax.ShapeDtypeStruct((NCORES, NP), jnp.float32),
                  jax.ShapeDtypeStruct((NCORES, NP), jnp.float32)],
        compiler_params=pltpu.CompilerParams(use_tc_tiling_on_sc=False,
                                             needs_layout_passes=False),
        scratch_types=[
            pltpu.VMEM((nchunks, CHUNK), jnp.int32),
            pltpu.VMEM((nchunks, CHUNK), jnp.int32),
            pltpu.VMEM((NP,), jnp.float32),
            pltpu.VMEM((NP,), jnp.float32),
            pltpu.VMEM((CHUNK,), jnp.float32),
            pltpu.VMEM((CHUNK,), jnp.float32),
            pltpu.VMEM((CHUNK,), jnp.float32),
            pltpu.VMEM((CHUNK,), jnp.float32),
            pltpu.VMEM((CHUNK,), jnp.float32),
            pltpu.SemaphoreType.DMA,
            pltpu.SemaphoreType.DMA,
            pltpu.SemaphoreType.DMA,
            pltpu.SemaphoreType.DMA,
            pltpu.VMEM_SHARED((NP,), jnp.float32),
            pltpu.VMEM_SHARED((NP,), jnp.float32),
        ],
    )
    def gs2_kernel(g0_hbm, g1_hbm, src_hbm, dst_hbm, out0_hbm, out1_hbm,
                   sidx, didx, tab0, tab1, v00, v10, v01, v11, stage,
                   sa0, sa1, sb0, sb1, acc0, acc1):
        c = lax.axis_index("c")
        s = lax.axis_index("s")
        t = c * NSUB + s
        zero = jnp.zeros((16,), jnp.float32)
        for j in range(CHUNK // 16):
            stage[pl.ds(j * 16, 16)] = zero

        def zbody(k, carry):
            off = s * rows_per_tile + k * CHUNK
            pltpu.sync_copy(stage, acc0.at[pl.ds(off, CHUNK)])
            pltpu.sync_copy(stage, acc1.at[pl.ds(off, CHUNK)])
            return carry

        lax.fori_loop(0, nz, zbody, 0)
        plsc.subcore_barrier()
        pltpu.sync_copy(src_hbm.at[t], sidx)
        pltpu.sync_copy(dst_hbm.at[t], didx)
        pltpu.sync_copy(g0_hbm, tab0)
        pltpu.sync_copy(g1_hbm, tab1)

        def compute(i, w0, w1):
            def group(j, carry2):
                idx = sidx[i, pl.ds(j * 16, 16)]
                w0[pl.ds(j * 16, 16)] = plsc.load_gather(tab0, [idx])
                w1[pl.ds(j * 16, 16)] = plsc.load_gather(tab1, [idx])
                return carry2

            lax.fori_loop(0, CHUNK // 16, group, 0)

        compute(0, v00, v10)
        pltpu.async_copy(v00, acc0.at[didx.at[0]], sa0, add=True)
        pltpu.async_copy(v10, acc1.at[didx.at[0]], sa1, add=True)
        compute(1, v01, v11)
        pltpu.async_copy(v01, acc0.at[didx.at[1]], sb0, add=True)
        pltpu.async_copy(v11, acc1.at[didx.at[1]], sb1, add=True)

        def body(ii, carry):
            i0 = 2 * ii
            i1 = 2 * ii + 1
            pltpu.make_async_copy(v00, acc0.at[didx.at[0]], sa0).wait()
            pltpu.make_async_copy(v10, acc1.at[didx.at[0]], sa1).wait()
            compute(i0, v00, v10)
            pltpu.async_copy(v00, acc0.at[didx.at[i0]], sa0, add=True)
            pltpu.async_copy(v10, acc1.at[didx.at[i0]], sa1, add=True)
            pltpu.make_async_copy(v01, acc0.at[didx.at[0]], sb0).wait()
            pltpu.make_async_copy(v11, acc1.at[didx.at[0]], sb1).wait()
            compute(i1, v01, v11)
            pltpu.async_copy(v01, acc0.at[didx.at[i1]], sb0, add=True)
            pltpu.async_copy(v11, acc1.at[didx.at[i1]], sb1, add=True)
            return carry

        lax.fori_loop(1, (nchunks - 1) // 2, body, 0)
        pltpu.make_async_copy(v00, acc0.at[didx.at[0]], sa0).wait()
        pltpu.make_async_copy(v10, acc1.at[didx.at[0]], sa1).wait()
        compute(nchunks - 1, v00, v10)
        pltpu.sync_copy(v00, acc0.at[didx.at[nchunks - 1]], add=True)
        pltpu.sync_copy(v10, acc1.at[didx.at[nchunks - 1]], add=True)
        pltpu.make_async_copy(v01, acc0.at[didx.at[0]], sb0).wait()
        pltpu.make_async_copy(v11, acc1.at[didx.at[0]], sb1).wait()
        plsc.subcore_barrier()

        def obody(k, carry):
            off = s * rows_per_tile + k * CHUNK
            pltpu.sync_copy(acc0.at[pl.ds(off, CHUNK)], stage)
            pltpu.sync_copy(stage, out0_hbm.at[c, pl.ds(off, CHUNK)])
            pltpu.sync_copy(acc1.at[pl.ds(off, CHUNK)], stage)
            pltpu.sync_copy(stage, out1_hbm.at[c, pl.ds(off, CHUNK)])
            return carry

        lax.fori_loop(0, nz, obody, 0)

    return gs2_kernel


def _tc_embed(degT, x_p, W1):
    D, H = W1.shape

    def body(deg_ref, x_ref, w_ref, o_ref):
        dinv = lax.rsqrt(deg_ref[:, 0:1] + deg_ref[:, 1:2] + 1.0)
        h = jnp.dot(x_ref[...], w_ref[...], preferred_element_type=jnp.float32)
        o_ref[...] = h * dinv

    return pl.pallas_call(
        body,
        grid=(NP // BN,),
        in_specs=[pl.BlockSpec((BN, 2), lambda i: (i, 0)),
                  pl.BlockSpec((BN, D), lambda i: (i, 0)),
                  pl.BlockSpec((D, H), lambda i: (0, 0))],
        out_specs=pl.BlockSpec((BN, H), lambda i: (i, 0)),
        out_shape=jax.ShapeDtypeStruct((NP, H), jnp.float32),
    )(degT, x_p, W1)


def _tc_layer2(degT, deg_p, P1, hs, b1r, W2):
    H = W2.shape[0]
    C = W2.shape[1]

    def body(degc_ref, degr_ref, p_ref, hs_ref, b1_ref, w2_ref, o_ref):
        dinv_c = lax.rsqrt(degc_ref[:, 0:1] + degc_ref[:, 1:2] + 1.0)
        dinv_r = lax.rsqrt(degr_ref[0:1, :] + degr_ref[1:2, :] + 1.0)
        pre = dinv_c * (p_ref[0] + p_ref[1] + hs_ref[...]) + b1_ref[...]
        h1 = jnp.maximum(pre, 0.0)
        gT = lax.dot_general(w2_ref[...], h1, (((0,), (1,)), ((), ())),
                             preferred_element_type=jnp.float32)
        o_ref[...] = gT * dinv_r

    return pl.pallas_call(
        body,
        grid=(NP // BN,),
        in_specs=[pl.BlockSpec((BN, 2), lambda i: (i, 0)),
                  pl.BlockSpec((2, BN), lambda i: (0, i)),
                  pl.BlockSpec((2, BN, H), lambda i: (0, i, 0)),
                  pl.BlockSpec((BN, H), lambda i: (i, 0)),
                  pl.BlockSpec((1, H), lambda i: (0, 0)),
                  pl.BlockSpec((H, C), lambda i: (0, 0))],
        out_specs=pl.BlockSpec((C, BN), lambda i: (0, i)),
        out_shape=jax.ShapeDtypeStruct((C, NP), jnp.float32),
    )(degT, deg_p, P1, hs, b1r, W2)


def _tc_logsoftmax(deg_p, p20, p21, gsT, b2c):
    def body(deg_ref, p0_ref, p1_ref, gs_ref, b2_ref, o_ref):
        dinv = lax.rsqrt(deg_ref[0:1, :] + deg_ref[1:2, :] + 1.0)
        z0 = dinv * (p0_ref[0:1, :] + p0_ref[1:2, :] + gs_ref[0:1, :]) + b2_ref[0, 0]
        z1 = dinv * (p1_ref[0:1, :] + p1_ref[1:2, :] + gs_ref[1:2, :]) + b2_ref[1, 0]
        m = jnp.maximum(z0, z1)
        lse = m + jnp.log(jnp.exp(z0 - m) + jnp.exp(z1 - m))
        o_ref[0:1, :] = z0 - lse
        o_ref[1:2, :] = z1 - lse

    return pl.pallas_call(
        body,
        grid=(NP // LN,),
        in_specs=[pl.BlockSpec((2, LN), lambda i: (0, i)),
                  pl.BlockSpec((2, LN), lambda i: (0, i)),
                  pl.BlockSpec((2, LN), lambda i: (0, i)),
                  pl.BlockSpec((2, LN), lambda i: (0, i)),
                  pl.BlockSpec((2, 1), lambda i: (0, 0))],
        out_specs=pl.BlockSpec((2, LN), lambda i: (0, i)),
        out_shape=jax.ShapeDtypeStruct((2, NP), jnp.float32),
    )(deg_p, p20, p21, gsT, b2c)


def kernel(x, edge_index, W1, b1, W2, b2):
    N, D = x.shape
    E = edge_index.shape[1]
    H = W1.shape[1]
    C = W2.shape[1]

    nchunks = E // NTILES // CHUNK
    nper = E // NTILES
    nmain = nper // 128
    src2d = edge_index[0].reshape(NTILES, nper)
    dst2d = edge_index[1].reshape(NTILES, nper)
    srcm = src2d[:, :nmain * 128].reshape(NTILES, nmain, 128)
    dstm = dst2d[:, :nmain * 128].reshape(NTILES, nmain, 128)
    srct = src2d[:, nmain * 128:]
    dstt = dst2d[:, nmain * 128:]
    src3d = edge_index[0].reshape(NTILES, nchunks, CHUNK)
    dst3d = edge_index[1].reshape(NTILES, nchunks, CHUNK)
    x_p = jnp.pad(x, ((0, NP - N), (0, 0)))
    b1r = b1.reshape(1, H)
    b2c = b2.reshape(C, 1)

    deg_p = _sc_degree(E)(dst3d).reshape(NCORES, NP)
    degT = deg_p.T
    hs = _tc_embed(degT, x_p, W1)
    P1 = _sc_gather_scatter_rows(E, H)(hs, srcm, dstm, srct, dstt)
    gsT = _tc_layer2(degT, deg_p, P1, hs, b1r, W2)
    g0 = gsT[0]
    g1 = gsT[1]
    p20, p21 = _sc_gather_scatter_elems(E)(g0, g1, src3d, dst3d)
    out2 = _tc_logsoftmax(deg_p, p20, p21, gsT, b2c)
    return out2.T[:N]


# speedup vs baseline: 1.0400x; 1.0400x over previous
import functools

import jax
import jax.numpy as jnp
from jax import lax
from jax.experimental import pallas as pl
from jax.experimental.pallas import tpu as pltpu
from jax.experimental.pallas import tpu_sc as plsc

NP = 10240
NCORES = 2
NSUB = 16
NTILES = NCORES * NSUB
CHUNK = 80
BN = 2560
LN = 2048


def _sc_degree(E):
    nchunks = E // NTILES // CHUNK
    rows_per_tile = NP // NSUB
    nz = rows_per_tile // CHUNK
    mesh = plsc.VectorSubcoreMesh(core_axis_name="c", subcore_axis_name="s")

    @functools.partial(
        pl.kernel,
        mesh=mesh,
        out_type=jax.ShapeDtypeStruct((NCORES * NP,), jnp.float32),
        compiler_params=pltpu.CompilerParams(use_tc_tiling_on_sc=False),
        scratch_types=[
            pltpu.VMEM((nchunks, CHUNK), jnp.int32),
            pltpu.VMEM((CHUNK,), jnp.float32),
            pltpu.VMEM((CHUNK,), jnp.float32),
            pltpu.SemaphoreType.DMA,
            pltpu.VMEM_SHARED((NP,), jnp.float32),
        ],
    )
    def deg_kernel(dst_hbm, out_hbm, didx, ones, stage, sem, acc):
        c = lax.axis_index("c")
        s = lax.axis_index("s")
        t = c * NSUB + s
        one = jnp.ones((16,), jnp.float32)
        zero = jnp.zeros((16,), jnp.float32)
        for j in range(CHUNK // 16):
            ones[pl.ds(j * 16, 16)] = one
            stage[pl.ds(j * 16, 16)] = zero

        def zbody(k, carry):
            pltpu.sync_copy(stage, acc.at[pl.ds(s * rows_per_tile + k * CHUNK, CHUNK)])
            return carry

        lax.fori_loop(0, nz, zbody, 0)
        plsc.subcore_barrier()
        pltpu.sync_copy(dst_hbm.at[t], didx)

        burst = 5

        def body(ii, carry):
            for b in range(burst):
                pltpu.async_copy(ones, acc.at[didx.at[burst * ii + b]], sem,
                                 add=True)
            for b in range(burst):
                pltpu.make_async_copy(ones, acc.at[didx.at[0]], sem).wait()
            return carry

        lax.fori_loop(0, nchunks // burst, body, 0)
        for k in range(nchunks - burst * (nchunks // burst)):
            pltpu.sync_copy(ones, acc.at[didx.at[burst * (nchunks // burst) + k]],
                            add=True)
        plsc.subcore_barrier()

        def obody(k, carry):
            off = s * rows_per_tile + k * CHUNK
            pltpu.sync_copy(acc.at[pl.ds(off, CHUNK)], stage)
            pltpu.sync_copy(stage, out_hbm.at[pl.ds(c * NP + off, CHUNK)])
            return carry

        lax.fori_loop(0, nz, obody, 0)

    return deg_kernel


def _sc_gather_scatter_rows(E, W):
    nchunks = E // NTILES // CHUNK
    rows_per_tile = NP // NSUB
    nz = rows_per_tile // CHUNK
    NBUF = 8
    mesh = plsc.VectorSubcoreMesh(core_axis_name="c", subcore_axis_name="s")

    @functools.partial(
        pl.kernel,
        mesh=mesh,
        out_type=jax.ShapeDtypeStruct((NCORES, NP, W), jnp.float32),
        compiler_params=pltpu.CompilerParams(use_tc_tiling_on_sc=False),
        scratch_types=(
            [pltpu.VMEM((nchunks, CHUNK), jnp.int32),
             pltpu.VMEM((nchunks, CHUNK), jnp.int32)]
            + [pltpu.VMEM((CHUNK, W), jnp.float32)] * (NBUF + 1)
            + [pltpu.SemaphoreType.DMA] * (2 * NBUF)
            + [pltpu.VMEM_SHARED((NP, W), jnp.float32)]
        ),
    )
    def gs_kernel(table_hbm, src_hbm, dst_hbm, out_hbm,
                  sidx, didx, r0, r1, r2, r3, r4, r5, r6, r7, stage,
                  g0, g1, g2, g3, g4, g5, g6, g7,
                  s0, s1, s2, s3, s4, s5, s6, s7, acc):
        rows = (r0, r1, r2, r3, r4, r5, r6, r7)
        gsem = (g0, g1, g2, g3, g4, g5, g6, g7)
        ssem = (s0, s1, s2, s3, s4, s5, s6, s7)
        nfull = nchunks // NBUF
        rem = nchunks - nfull * NBUF
        c = lax.axis_index("c")
        s = lax.axis_index("s")
        t = c * NSUB + s
        zero = jnp.zeros((16,), jnp.float32)

        def zrow(i, carry):
            for j in range(W // 16):
                stage[i, pl.ds(j * 16, 16)] = zero
            return carry

        lax.fori_loop(0, CHUNK, zrow, 0)

        def zbody(k, carry):
            pltpu.sync_copy(stage, acc.at[pl.ds(s * rows_per_tile + k * CHUNK, CHUNK)])
            return carry

        lax.fori_loop(0, nz, zbody, 0)
        plsc.subcore_barrier()
        pltpu.sync_copy(src_hbm.at[t], sidx)
        pltpu.sync_copy(dst_hbm.at[t], didx)

        for b in range(NBUF):
            pltpu.async_copy(table_hbm.at[sidx.at[b]], rows[b], gsem[b])
        for b in range(NBUF):
            pltpu.make_async_copy(table_hbm.at[sidx.at[b]], rows[b], gsem[b]).wait()
            pltpu.async_copy(rows[b], acc.at[didx.at[b]], ssem[b], add=True)

        def body(ii, carry):
            base = ii * NBUF
            for b in range(NBUF):
                pltpu.make_async_copy(rows[b], acc.at[didx.at[0]], ssem[b]).wait()
                pltpu.async_copy(table_hbm.at[sidx.at[base + b]], rows[b], gsem[b])
            for b in range(NBUF):
                pltpu.make_async_copy(table_hbm.at[sidx.at[base + b]],
                                      rows[b], gsem[b]).wait()
                pltpu.async_copy(rows[b], acc.at[didx.at[base + b]], ssem[b],
                                 add=True)
            return carry

        lax.fori_loop(1, nfull, body, 0)
        for b in range(NBUF):
            pltpu.make_async_copy(rows[b], acc.at[didx.at[0]], ssem[b]).wait()
        for k in range(rem):
            i = nfull * NBUF + k
            pltpu.async_copy(table_hbm.at[sidx.at[i]], rows[k], gsem[k])
        for k in range(rem):
            i = nfull * NBUF + k
            pltpu.make_async_copy(table_hbm.at[sidx.at[i]], rows[k], gsem[k]).wait()
            pltpu.async_copy(rows[k], acc.at[didx.at[i]], ssem[k], add=True)
        for k in range(rem):
            pltpu.make_async_copy(rows[k], acc.at[didx.at[0]], ssem[k]).wait()
        plsc.subcore_barrier()

        def obody(k, carry):
            off = s * rows_per_tile + k * CHUNK
            pltpu.sync_copy(acc.at[pl.ds(off, CHUNK)], stage)
            pltpu.sync_copy(stage, out_hbm.at[c, pl.ds(off, CHUNK)])
            return carry

        lax.fori_loop(0, nz, obody, 0)

    return gs_kernel


def _sc_gather_scatter_elems(E):
    nchunks = E // NTILES // CHUNK
    rows_per_tile = NP // NSUB
    nz = rows_per_tile // CHUNK
    mesh = plsc.VectorSubcoreMesh(core_axis_name="c", subcore_axis_name="s")

    @functools.partial(
        pl.kernel,
        mesh=mesh,
        out_type=[jax.ShapeDtypeStruct((NCORES, NP), jnp.float32),
                  jax.ShapeDtypeStruct((NCORES, NP), jnp.float32)],
        compiler_params=pltpu.CompilerParams(use_tc_tiling_on_sc=False,
                                             needs_layout_passes=False),
        scratch_types=[
            pltpu.VMEM((nchunks, CHUNK), jnp.int32),
            pltpu.VMEM((nchunks, CHUNK), jnp.int32),
            pltpu.VMEM((NP,), jnp.float32),
            pltpu.VMEM((NP,), jnp.float32),
            pltpu.VMEM((CHUNK,), jnp.float32),
            pltpu.VMEM((CHUNK,), jnp.float32),
            pltpu.VMEM((CHUNK,), jnp.float32),
            pltpu.VMEM((CHUNK,), jnp.float32),
            pltpu.VMEM((CHUNK,), jnp.float32),
            pltpu.SemaphoreType.DMA,
            pltpu.SemaphoreType.DMA,
            pltpu.SemaphoreType.DMA,
            pltpu.SemaphoreType.DMA,
            pltpu.VMEM_SHARED((NP,), jnp.float32),
            pltpu.VMEM_SHARED((NP,), jnp.float32),
        ],
    )
    def gs2_kernel(g0_hbm, g1_hbm, src_hbm, dst_hbm, out0_hbm, out1_hbm,
                   sidx, didx, tab0, tab1, v00, v10, v01, v11, stage,
                   sa0, sa1, sb0, sb1, acc0, acc1):
        c = lax.axis_index("c")
        s = lax.axis_index("s")
        t = c * NSUB + s
        zero = jnp.zeros((16,), jnp.float32)
        for j in range(CHUNK // 16):
            stage[pl.ds(j * 16, 16)] = zero

        def zbody(k, carry):
            off = s * rows_per_tile + k * CHUNK
            pltpu.sync_copy(stage, acc0.at[pl.ds(off, CHUNK)])
            pltpu.sync_copy(stage, acc1.at[pl.ds(off, CHUNK)])
            return carry

        lax.fori_loop(0, nz, zbody, 0)
        plsc.subcore_barrier()
        pltpu.sync_copy(src_hbm.at[t], sidx)
        pltpu.sync_copy(dst_hbm.at[t], didx)
        pltpu.sync_copy(g0_hbm, tab0)
        pltpu.sync_copy(g1_hbm, tab1)

        def compute(i, w0, w1):
            def group(j, carry2):
                idx = sidx[i, pl.ds(j * 16, 16)]
                w0[pl.ds(j * 16, 16)] = plsc.load_gather(tab0, [idx])
                w1[pl.ds(j * 16, 16)] = plsc.load_gather(tab1, [idx])
                return carry2

            lax.fori_loop(0, CHUNK // 16, group, 0)

        compute(0, v00, v10)
        pltpu.async_copy(v00, acc0.at[didx.at[0]], sa0, add=True)
        pltpu.async_copy(v10, acc1.at[didx.at[0]], sa1, add=True)
        compute(1, v01, v11)
        pltpu.async_copy(v01, acc0.at[didx.at[1]], sb0, add=True)
        pltpu.async_copy(v11, acc1.at[didx.at[1]], sb1, add=True)

        def body(ii, carry):
            i0 = 2 * ii
            i1 = 2 * ii + 1
            pltpu.make_async_copy(v00, acc0.at[didx.at[0]], sa0).wait()
            pltpu.make_async_copy(v10, acc1.at[didx.at[0]], sa1).wait()
            compute(i0, v00, v10)
            pltpu.async_copy(v00, acc0.at[didx.at[i0]], sa0, add=True)
            pltpu.async_copy(v10, acc1.at[didx.at[i0]], sa1, add=True)
            pltpu.make_async_copy(v01, acc0.at[didx.at[0]], sb0).wait()
            pltpu.make_async_copy(v11, acc1.at[didx.at[0]], sb1).wait()
            compute(i1, v01, v11)
            pltpu.async_copy(v01, acc0.at[didx.at[i1]], sb0, add=True)
            pltpu.async_copy(v11, acc1.at[didx.at[i1]], sb1, add=True)
            return carry

        lax.fori_loop(1, (nchunks - 1) // 2, body, 0)
        pltpu.make_async_copy(v00, acc0.at[didx.at[0]], sa0).wait()
        pltpu.make_async_copy(v10, acc1.at[didx.at[0]], sa1).wait()
        compute(nchunks - 1, v00, v10)
        pltpu.sync_copy(v00, acc0.at[didx.at[nchunks - 1]], add=True)
        pltpu.sync_copy(v10, acc1.at[didx.at[nchunks - 1]], add=True)
        pltpu.make_async_copy(v01, acc0.at[didx.at[0]], sb0).wait()
        pltpu.make_async_copy(v11, acc1.at[didx.at[0]], sb1).wait()
        plsc.subcore_barrier()

        def obody(k, carry):
            off = s * rows_per_tile + k * CHUNK
            pltpu.sync_copy(acc0.at[pl.ds(off, CHUNK)], stage)
            pltpu.sync_copy(stage, out0_hbm.at[c, pl.ds(off, CHUNK)])
            pltpu.sync_copy(acc1.at[pl.ds(off, CHUNK)], stage)
            pltpu.sync_copy(stage, out1_hbm.at[c, pl.ds(off, CHUNK)])
            return carry

        lax.fori_loop(0, nz, obody, 0)

    return gs2_kernel


def _tc_embed(degT, x_p, W1):
    D, H = W1.shape

    def body(deg_ref, x_ref, w_ref, o_ref):
        dinv = lax.rsqrt(deg_ref[:, 0:1] + deg_ref[:, 1:2] + 1.0)
        h = jnp.dot(x_ref[...], w_ref[...], preferred_element_type=jnp.float32)
        o_ref[...] = h * dinv

    return pl.pallas_call(
        body,
        grid=(NP // BN,),
        in_specs=[pl.BlockSpec((BN, 2), lambda i: (i, 0)),
                  pl.BlockSpec((BN, D), lambda i: (i, 0)),
                  pl.BlockSpec((D, H), lambda i: (0, 0))],
        out_specs=pl.BlockSpec((BN, H), lambda i: (i, 0)),
        out_shape=jax.ShapeDtypeStruct((NP, H), jnp.float32),
    )(degT, x_p, W1)


def _tc_layer2(degT, deg_p, P1, hs, b1r, W2):
    H = W2.shape[0]
    C = W2.shape[1]

    def body(degc_ref, degr_ref, p_ref, hs_ref, b1_ref, w2_ref, o_ref):
        dinv_c = lax.rsqrt(degc_ref[:, 0:1] + degc_ref[:, 1:2] + 1.0)
        dinv_r = lax.rsqrt(degr_ref[0:1, :] + degr_ref[1:2, :] + 1.0)
        pre = dinv_c * (p_ref[0] + p_ref[1] + hs_ref[...]) + b1_ref[...]
        h1 = jnp.maximum(pre, 0.0)
        gT = lax.dot_general(w2_ref[...], h1, (((0,), (1,)), ((), ())),
                             preferred_element_type=jnp.float32)
        o_ref[...] = gT * dinv_r

    return pl.pallas_call(
        body,
        grid=(NP // BN,),
        in_specs=[pl.BlockSpec((BN, 2), lambda i: (i, 0)),
                  pl.BlockSpec((2, BN), lambda i: (0, i)),
                  pl.BlockSpec((2, BN, H), lambda i: (0, i, 0)),
                  pl.BlockSpec((BN, H), lambda i: (i, 0)),
                  pl.BlockSpec((1, H), lambda i: (0, 0)),
                  pl.BlockSpec((H, C), lambda i: (0, 0))],
        out_specs=pl.BlockSpec((C, BN), lambda i: (0, i)),
        out_shape=jax.ShapeDtypeStruct((C, NP), jnp.float32),
    )(degT, deg_p, P1, hs, b1r, W2)


def _tc_logsoftmax(deg_p, p20, p21, gsT, b2c):
    def body(deg_ref, p0_ref, p1_ref, gs_ref, b2_ref, o_ref):
        dinv = lax.rsqrt(deg_ref[0:1, :] + deg_ref[1:2, :] + 1.0)
        z0 = dinv * (p0_ref[0:1, :] + p0_ref[1:2, :] + gs_ref[0:1, :]) + b2_ref[0, 0]
        z1 = dinv * (p1_ref[0:1, :] + p1_ref[1:2, :] + gs_ref[1:2, :]) + b2_ref[1, 0]
        m = jnp.maximum(z0, z1)
        lse = m + jnp.log(jnp.exp(z0 - m) + jnp.exp(z1 - m))
        o_ref[0:1, :] = z0 - lse
        o_ref[1:2, :] = z1 - lse

    return pl.pallas_call(
        body,
        grid=(NP // LN,),
        in_specs=[pl.BlockSpec((2, LN), lambda i: (0, i)),
                  pl.BlockSpec((2, LN), lambda i: (0, i)),
                  pl.BlockSpec((2, LN), lambda i: (0, i)),
                  pl.BlockSpec((2, LN), lambda i: (0, i)),
                  pl.BlockSpec((2, 1), lambda i: (0, 0))],
        out_specs=pl.BlockSpec((2, LN), lambda i: (0, i)),
        out_shape=jax.ShapeDtypeStruct((2, NP), jnp.float32),
    )(deg_p, p20, p21, gsT, b2c)


def kernel(x, edge_index, W1, b1, W2, b2):
    N, D = x.shape
    E = edge_index.shape[1]
    H = W1.shape[1]
    C = W2.shape[1]

    nchunks = E // NTILES // CHUNK
    src3d = edge_index[0].reshape(NTILES, nchunks, CHUNK)
    dst3d = edge_index[1].reshape(NTILES, nchunks, CHUNK)
    x_p = jnp.pad(x, ((0, NP - N), (0, 0)))
    b1r = b1.reshape(1, H)
    b2c = b2.reshape(C, 1)

    deg_p = _sc_degree(E)(dst3d).reshape(NCORES, NP)
    degT = deg_p.T
    hs = _tc_embed(degT, x_p, W1)
    P1 = _sc_gather_scatter_rows(E, H)(hs, src3d, dst3d)
    gsT = _tc_layer2(degT, deg_p, P1, hs, b1r, W2)
    g0 = gsT[0]
    g1 = gsT[1]
    p20, p21 = _sc_gather_scatter_elems(E)(g0, g1, src3d, dst3d)
    out2 = _tc_logsoftmax(deg_p, p20, p21, gsT, b2c)
    return out2.T[:N]


# speedup vs baseline: 1.0463x; 1.0060x over previous
import functools

import jax
import jax.numpy as jnp
from jax import lax
from jax.experimental import pallas as pl
from jax.experimental.pallas import tpu as pltpu
from jax.experimental.pallas import tpu_sc as plsc

NP = 10240
NCORES = 2
NSUB = 16
NTILES = NCORES * NSUB
CHUNK = 80
BN = 2560
LN = 2048


def _sc_degree(E):
    nchunks = E // NTILES // CHUNK
    rows_per_tile = NP // NSUB
    nz = rows_per_tile // CHUNK
    mesh = plsc.VectorSubcoreMesh(core_axis_name="c", subcore_axis_name="s")

    @functools.partial(
        pl.kernel,
        mesh=mesh,
        out_type=jax.ShapeDtypeStruct((NCORES * NP,), jnp.float32),
        compiler_params=pltpu.CompilerParams(use_tc_tiling_on_sc=False),
        scratch_types=[
            pltpu.VMEM((nchunks, CHUNK), jnp.int32),
            pltpu.VMEM((CHUNK,), jnp.float32),
            pltpu.VMEM((CHUNK,), jnp.float32),
            pltpu.SemaphoreType.DMA,
            pltpu.VMEM_SHARED((NP,), jnp.float32),
        ],
    )
    def deg_kernel(dst_hbm, out_hbm, didx, ones, stage, sem, acc):
        c = lax.axis_index("c")
        s = lax.axis_index("s")
        t = c * NSUB + s
        one = jnp.ones((16,), jnp.float32)
        zero = jnp.zeros((16,), jnp.float32)
        for j in range(CHUNK // 16):
            ones[pl.ds(j * 16, 16)] = one
            stage[pl.ds(j * 16, 16)] = zero

        def zbody(k, carry):
            pltpu.sync_copy(stage, acc.at[pl.ds(s * rows_per_tile + k * CHUNK, CHUNK)])
            return carry

        lax.fori_loop(0, nz, zbody, 0)
        plsc.subcore_barrier()
        pltpu.sync_copy(dst_hbm.at[t], didx)

        burst = 25

        def body(ii, carry):
            for b in range(burst):
                pltpu.async_copy(ones, acc.at[didx.at[burst * ii + b]], sem,
                                 add=True)
            for b in range(burst):
                pltpu.make_async_copy(ones, acc.at[didx.at[0]], sem).wait()
            return carry

        lax.fori_loop(0, nchunks // burst, body, 0)
        for k in range(nchunks - burst * (nchunks // burst)):
            pltpu.sync_copy(ones, acc.at[didx.at[burst * (nchunks // burst) + k]],
                            add=True)
        plsc.subcore_barrier()

        def obody(k, carry):
            off = s * rows_per_tile + k * CHUNK
            pltpu.sync_copy(acc.at[pl.ds(off, CHUNK)], stage)
            pltpu.sync_copy(stage, out_hbm.at[pl.ds(c * NP + off, CHUNK)])
            return carry

        lax.fori_loop(0, nz, obody, 0)

    return deg_kernel


def _sc_gather_scatter_rows(E, W):
    nchunks = E // NTILES // CHUNK
    rows_per_tile = NP // NSUB
    nz = rows_per_tile // CHUNK
    NBUF = 8
    mesh = plsc.VectorSubcoreMesh(core_axis_name="c", subcore_axis_name="s")

    @functools.partial(
        pl.kernel,
        mesh=mesh,
        out_type=jax.ShapeDtypeStruct((NCORES, NP, W), jnp.float32),
        compiler_params=pltpu.CompilerParams(use_tc_tiling_on_sc=False),
        scratch_types=(
            [pltpu.VMEM((nchunks, CHUNK), jnp.int32),
             pltpu.VMEM((nchunks, CHUNK), jnp.int32)]
            + [pltpu.VMEM((CHUNK, W), jnp.float32)] * (NBUF + 1)
            + [pltpu.SemaphoreType.DMA] * (2 * NBUF)
            + [pltpu.VMEM_SHARED((NP, W), jnp.float32)]
        ),
    )
    def gs_kernel(table_hbm, src_hbm, dst_hbm, out_hbm,
                  sidx, didx, r0, r1, r2, r3, r4, r5, r6, r7, stage,
                  g0, g1, g2, g3, g4, g5, g6, g7,
                  s0, s1, s2, s3, s4, s5, s6, s7, acc):
        rows = (r0, r1, r2, r3, r4, r5, r6, r7)
        gsem = (g0, g1, g2, g3, g4, g5, g6, g7)
        ssem = (s0, s1, s2, s3, s4, s5, s6, s7)
        nfull = nchunks // NBUF
        rem = nchunks - nfull * NBUF
        c = lax.axis_index("c")
        s = lax.axis_index("s")
        t = c * NSUB + s
        zero = jnp.zeros((16,), jnp.float32)

        def zrow(i, carry):
            for j in range(W // 16):
                stage[i, pl.ds(j * 16, 16)] = zero
            return carry

        lax.fori_loop(0, CHUNK, zrow, 0)

        def zbody(k, carry):
            pltpu.sync_copy(stage, acc.at[pl.ds(s * rows_per_tile + k * CHUNK, CHUNK)])
            return carry

        lax.fori_loop(0, nz, zbody, 0)
        plsc.subcore_barrier()
        pltpu.sync_copy(src_hbm.at[t], sidx)
        pltpu.sync_copy(dst_hbm.at[t], didx)

        for b in range(NBUF):
            pltpu.async_copy(table_hbm.at[sidx.at[b]], rows[b], gsem[b])
        for b in range(NBUF):
            pltpu.make_async_copy(table_hbm.at[sidx.at[b]], rows[b], gsem[b]).wait()
            pltpu.async_copy(rows[b], acc.at[didx.at[b]], ssem[b], add=True)

        def body(ii, carry):
            base = ii * NBUF
            for b in range(NBUF):
                pltpu.make_async_copy(rows[b], acc.at[didx.at[0]], ssem[b]).wait()
                pltpu.async_copy(table_hbm.at[sidx.at[base + b]], rows[b], gsem[b])
            for b in range(NBUF):
                pltpu.make_async_copy(table_hbm.at[sidx.at[base + b]],
                                      rows[b], gsem[b]).wait()
                pltpu.async_copy(rows[b], acc.at[didx.at[base + b]], ssem[b],
                                 add=True)
            return carry

        lax.fori_loop(1, nfull, body, 0)
        for b in range(NBUF):
            pltpu.make_async_copy(rows[b], acc.at[didx.at[0]], ssem[b]).wait()
        for k in range(rem):
            i = nfull * NBUF + k
            pltpu.async_copy(table_hbm.at[sidx.at[i]], rows[k], gsem[k])
        for k in range(rem):
            i = nfull * NBUF + k
            pltpu.make_async_copy(table_hbm.at[sidx.at[i]], rows[k], gsem[k]).wait()
            pltpu.async_copy(rows[k], acc.at[didx.at[i]], ssem[k], add=True)
        for k in range(rem):
            pltpu.make_async_copy(rows[k], acc.at[didx.at[0]], ssem[k]).wait()
        plsc.subcore_barrier()

        def obody(k, carry):
            off = s * rows_per_tile + k * CHUNK
            pltpu.sync_copy(acc.at[pl.ds(off, CHUNK)], stage)
            pltpu.sync_copy(stage, out_hbm.at[c, pl.ds(off, CHUNK)])
            return carry

        lax.fori_loop(0, nz, obody, 0)

    return gs_kernel


def _sc_gather_scatter_elems(E):
    nchunks = E // NTILES // CHUNK
    rows_per_tile = NP // NSUB
    nz = rows_per_tile // CHUNK
    mesh = plsc.VectorSubcoreMesh(core_axis_name="c", subcore_axis_name="s")

    @functools.partial(
        pl.kernel,
        mesh=mesh,
        out_type=[jax.ShapeDtypeStruct((NCORES, NP), jnp.float32),
                  jax.ShapeDtypeStruct((NCORES, NP), jnp.float32)],
        compiler_params=pltpu.CompilerParams(use_tc_tiling_on_sc=False,
                                             needs_layout_passes=False),
        scratch_types=[
            pltpu.VMEM((nchunks, CHUNK), jnp.int32),
            pltpu.VMEM((nchunks, CHUNK), jnp.int32),
            pltpu.VMEM((NP,), jnp.float32),
            pltpu.VMEM((NP,), jnp.float32),
            pltpu.VMEM((CHUNK,), jnp.float32),
            pltpu.VMEM((CHUNK,), jnp.float32),
            pltpu.VMEM((CHUNK,), jnp.float32),
            pltpu.VMEM((CHUNK,), jnp.float32),
            pltpu.VMEM((CHUNK,), jnp.float32),
            pltpu.SemaphoreType.DMA,
            pltpu.SemaphoreType.DMA,
            pltpu.SemaphoreType.DMA,
            pltpu.SemaphoreType.DMA,
            pltpu.VMEM_SHARED((NP,), jnp.float32),
            pltpu.VMEM_SHARED((NP,), jnp.float32),
        ],
    )
    def gs2_kernel(g0_hbm, g1_hbm, src_hbm, dst_hbm, out0_hbm, out1_hbm,
                   sidx, didx, tab0, tab1, v00, v10, v01, v11, stage,
                   sa0, sa1, sb0, sb1, acc0, acc1):
        c = lax.axis_index("c")
        s = lax.axis_index("s")
        t = c * NSUB + s
        zero = jnp.zeros((16,), jnp.float32)
        for j in range(CHUNK // 16):
            stage[pl.ds(j * 16, 16)] = zero

        def zbody(k, carry):
            off = s * rows_per_tile + k * CHUNK
            pltpu.sync_copy(stage, acc0.at[pl.ds(off, CHUNK)])
            pltpu.sync_copy(stage, acc1.at[pl.ds(off, CHUNK)])
            return carry

        lax.fori_loop(0, nz, zbody, 0)
        plsc.subcore_barrier()
        pltpu.sync_copy(src_hbm.at[t], sidx)
        pltpu.sync_copy(dst_hbm.at[t], didx)
        pltpu.sync_copy(g0_hbm, tab0)
        pltpu.sync_copy(g1_hbm, tab1)

        def compute(i, w0, w1):
            def group(j, carry2):
                idx = sidx[i, pl.ds(j * 16, 16)]
                w0[pl.ds(j * 16, 16)] = plsc.load_gather(tab0, [idx])
                w1[pl.ds(j * 16, 16)] = plsc.load_gather(tab1, [idx])
                return carry2

            lax.fori_loop(0, CHUNK // 16, group, 0)

        compute(0, v00, v10)
        pltpu.async_copy(v00, acc0.at[didx.at[0]], sa0, add=True)
        pltpu.async_copy(v10, acc1.at[didx.at[0]], sa1, add=True)
        compute(1, v01, v11)
        pltpu.async_copy(v01, acc0.at[didx.at[1]], sb0, add=True)
        pltpu.async_copy(v11, acc1.at[didx.at[1]], sb1, add=True)

        def body(ii, carry):
            i0 = 2 * ii
            i1 = 2 * ii + 1
            pltpu.make_async_copy(v00, acc0.at[didx.at[0]], sa0).wait()
            pltpu.make_async_copy(v10, acc1.at[didx.at[0]], sa1).wait()
            compute(i0, v00, v10)
            pltpu.async_copy(v00, acc0.at[didx.at[i0]], sa0, add=True)
            pltpu.async_copy(v10, acc1.at[didx.at[i0]], sa1, add=True)
            pltpu.make_async_copy(v01, acc0.at[didx.at[0]], sb0).wait()
            pltpu.make_async_copy(v11, acc1.at[didx.at[0]], sb1).wait()
            compute(i1, v01, v11)
            pltpu.async_copy(v01, acc0.at[didx.at[i1]], sb0, add=True)
            pltpu.async_copy(v11, acc1.at[didx.at[i1]], sb1, add=True)
            return carry

        lax.fori_loop(1, (nchunks - 1) // 2, body, 0)
        pltpu.make_async_copy(v00, acc0.at[didx.at[0]], sa0).wait()
        pltpu.make_async_copy(v10, acc1.at[didx.at[0]], sa1).wait()
        compute(nchunks - 1, v00, v10)
        pltpu.sync_copy(v00, acc0.at[didx.at[nchunks - 1]], add=True)
        pltpu.sync_copy(v10, acc1.at[didx.at[nchunks - 1]], add=True)
        pltpu.make_async_copy(v01, acc0.at[didx.at[0]], sb0).wait()
        pltpu.make_async_copy(v11, acc1.at[didx.at[0]], sb1).wait()
        plsc.subcore_barrier()

        def obody(k, carry):
            off = s * rows_per_tile + k * CHUNK
            pltpu.sync_copy(acc0.at[pl.ds(off, CHUNK)], stage)
            pltpu.sync_copy(stage, out0_hbm.at[c, pl.ds(off, CHUNK)])
            pltpu.sync_copy(acc1.at[pl.ds(off, CHUNK)], stage)
            pltpu.sync_copy(stage, out1_hbm.at[c, pl.ds(off, CHUNK)])
            return carry

        lax.fori_loop(0, nz, obody, 0)

    return gs2_kernel


def _tc_embed(degT, x_p, W1):
    D, H = W1.shape

    def body(deg_ref, x_ref, w_ref, o_ref):
        dinv = lax.rsqrt(deg_ref[:, 0:1] + deg_ref[:, 1:2] + 1.0)
        h = jnp.dot(x_ref[...], w_ref[...], preferred_element_type=jnp.float32)
        o_ref[...] = h * dinv

    return pl.pallas_call(
        body,
        grid=(NP // BN,),
        in_specs=[pl.BlockSpec((BN, 2), lambda i: (i, 0)),
                  pl.BlockSpec((BN, D), lambda i: (i, 0)),
                  pl.BlockSpec((D, H), lambda i: (0, 0))],
        out_specs=pl.BlockSpec((BN, H), lambda i: (i, 0)),
        out_shape=jax.ShapeDtypeStruct((NP, H), jnp.float32),
    )(degT, x_p, W1)


def _tc_layer2(degT, deg_p, P1, hs, b1r, W2):
    H = W2.shape[0]
    C = W2.shape[1]

    def body(degc_ref, degr_ref, p_ref, hs_ref, b1_ref, w2_ref, o_ref):
        dinv_c = lax.rsqrt(degc_ref[:, 0:1] + degc_ref[:, 1:2] + 1.0)
        dinv_r = lax.rsqrt(degr_ref[0:1, :] + degr_ref[1:2, :] + 1.0)
        pre = dinv_c * (p_ref[0] + p_ref[1] + hs_ref[...]) + b1_ref[...]
        h1 = jnp.maximum(pre, 0.0)
        gT = lax.dot_general(w2_ref[...], h1, (((0,), (1,)), ((), ())),
                             preferred_element_type=jnp.float32)
        o_ref[...] = gT * dinv_r

    return pl.pallas_call(
        body,
        grid=(NP // BN,),
        in_specs=[pl.BlockSpec((BN, 2), lambda i: (i, 0)),
                  pl.BlockSpec((2, BN), lambda i: (0, i)),
                  pl.BlockSpec((2, BN, H), lambda i: (0, i, 0)),
                  pl.BlockSpec((BN, H), lambda i: (i, 0)),
                  pl.BlockSpec((1, H), lambda i: (0, 0)),
                  pl.BlockSpec((H, C), lambda i: (0, 0))],
        out_specs=pl.BlockSpec((C, BN), lambda i: (0, i)),
        out_shape=jax.ShapeDtypeStruct((C, NP), jnp.float32),
    )(degT, deg_p, P1, hs, b1r, W2)


def _tc_logsoftmax(deg_p, p20, p21, gsT, b2c):
    def body(deg_ref, p0_ref, p1_ref, gs_ref, b2_ref, o_ref):
        dinv = lax.rsqrt(deg_ref[0:1, :] + deg_ref[1:2, :] + 1.0)
        z0 = dinv * (p0_ref[0:1, :] + p0_ref[1:2, :] + gs_ref[0:1, :]) + b2_ref[0, 0]
        z1 = dinv * (p1_ref[0:1, :] + p1_ref[1:2, :] + gs_ref[1:2, :]) + b2_ref[1, 0]
        m = jnp.maximum(z0, z1)
        lse = m + jnp.log(jnp.exp(z0 - m) + jnp.exp(z1 - m))
        o_ref[0:1, :] = z0 - lse
        o_ref[1:2, :] = z1 - lse

    return pl.pallas_call(
        body,
        grid=(NP // LN,),
        in_specs=[pl.BlockSpec((2, LN), lambda i: (0, i)),
                  pl.BlockSpec((2, LN), lambda i: (0, i)),
                  pl.BlockSpec((2, LN), lambda i: (0, i)),
                  pl.BlockSpec((2, LN), lambda i: (0, i)),
                  pl.BlockSpec((2, 1), lambda i: (0, 0))],
        out_specs=pl.BlockSpec((2, LN), lambda i: (0, i)),
        out_shape=jax.ShapeDtypeStruct((2, NP), jnp.float32),
    )(deg_p, p20, p21, gsT, b2c)


def kernel(x, edge_index, W1, b1, W2, b2):
    N, D = x.shape
    E = edge_index.shape[1]
    H = W1.shape[1]
    C = W2.shape[1]

    nchunks = E // NTILES // CHUNK
    src3d = edge_index[0].reshape(NTILES, nchunks, CHUNK)
    dst3d = edge_index[1].reshape(NTILES, nchunks, CHUNK)
    x_p = jnp.pad(x, ((0, NP - N), (0, 0)))
    b1r = b1.reshape(1, H)
    b2c = b2.reshape(C, 1)

    deg_p = _sc_degree(E)(dst3d).reshape(NCORES, NP)
    degT = deg_p.T
    hs = _tc_embed(degT, x_p, W1)
    P1 = _sc_gather_scatter_rows(E, H)(hs, src3d, dst3d)
    gsT = _tc_layer2(degT, deg_p, P1, hs, b1r, W2)
    g0 = gsT[0]
    g1 = gsT[1]
    p20, p21 = _sc_gather_scatter_elems(E)(g0, g1, src3d, dst3d)
    out2 = _tc_logsoftmax(deg_p, p20, p21, gsT, b2c)
    return out2.T[:N]
